# Initial kernel scaffold; baseline (speedup 1.0000x reference)
import jax
import jax.numpy as jnp
from jax.experimental import pallas as pl


def kernel(xyz1, xyz2):
    raise NotImplementedError("write your pallas kernel here")


# speedup vs baseline: 1.5748x; 1.5748x over previous
import functools

import jax
import jax.numpy as jnp
from jax.experimental import pallas as pl


B, N, M, C = 2, 4096, 4096, 3
BN = 512


def _chamfer_body(x1_ref, x2_ref, d1_ref, d2_ref):
    nb = pl.program_id(1)
    acc = None
    for c in range(C):
        a = x1_ref[0, :, c : c + 1]
        b = x2_ref[0, c : c + 1, :]
        diff = a - b
        sq = diff * diff
        acc = sq if acc is None else acc + sq
    d1_ref[0, 0, :] = jnp.min(acc, axis=1)
    part = jnp.min(acc, axis=0)

    @pl.when(nb == 0)
    def _init():
        d2_ref[0, 0, :] = part

    @pl.when(nb != 0)
    def _accum():
        d2_ref[0, 0, :] = jnp.minimum(d2_ref[0, 0, :], part)


@jax.jit
def kernel(xyz1, xyz2):
    x2t = jnp.transpose(xyz2, (0, 2, 1))
    grid = (B, N // BN)
    d1, d2 = pl.pallas_call(
        _chamfer_body,
        grid=grid,
        in_specs=[
            pl.BlockSpec((1, BN, C), lambda b, nb: (b, nb, 0)),
            pl.BlockSpec((1, C, M), lambda b, nb: (b, 0, 0)),
        ],
        out_specs=[
            pl.BlockSpec((1, 1, BN), lambda b, nb: (b, 0, nb)),
            pl.BlockSpec((1, 1, M), lambda b, nb: (b, 0, 0)),
        ],
        out_shape=[
            jax.ShapeDtypeStruct((B, 1, N), jnp.float32),
            jax.ShapeDtypeStruct((B, 1, M), jnp.float32),
        ],
    )(xyz1, x2t)
    return d1.reshape(B, N), d2.reshape(B, M)
